# baseline (speedup 1.0000x reference)
import functools

import jax
import jax.numpy as jnp
from jax import lax
from jax.experimental import pallas as pl
from jax.experimental.pallas import tpu as pltpu
from jax.experimental.pallas import tpu_sc as plsc

EMB = 128
BATCH = 4096
NEG = 20
L = 16
NV = EMB // L
NC, NS = 2, 16
NW = NC * NS
BPW = BATCH // NW
BB = 8
NT = BPW // BB
NROWS = BB * NEG
CHUNK = 80
NCH = NROWS // CHUNK


def _sc_scores(ctr_ids, ctx_ids, neg_flat, u_global, v_global, u_reg, v_reg):
    mesh = plsc.VectorSubcoreMesh(
        core_axis_name="c", subcore_axis_name="s", num_cores=NC, num_subcores=NS
    )

    row_buf = lambda n: pltpu.VMEM((n, EMB), jnp.float32)
    set_bufs = [row_buf(BB), row_buf(BB), row_buf(NROWS)]

    @functools.partial(
        pl.kernel,
        out_type=jax.ShapeDtypeStruct((1 + NEG, BATCH), jnp.float32),
        mesh=mesh,
        compiler_params=pltpu.CompilerParams(needs_layout_passes=False),
        scratch_types=[
            pltpu.VMEM((BPW,), jnp.int32),
            pltpu.VMEM((BPW,), jnp.int32),
            pltpu.VMEM((BPW * NEG,), jnp.int32),
            *set_bufs,
            *set_bufs,
            pltpu.VMEM((1 + NEG, BPW), jnp.float32),
            pltpu.SemaphoreType.DMA,
            pltpu.SemaphoreType.DMA,
        ],
    )
    def body(ctr_hbm, ctx_hbm, neg_hbm, ug_hbm, vg_hbm, ur_hbm, vr_hbm, out_hbm,
             ctr_idx, ctx_idx, neg_idx, *rest):
        bufs = (rest[0:3], rest[3:6])
        scores = rest[6]
        sems = (rest[7], rest[8])
        wid = lax.axis_index("s") * NC + lax.axis_index("c")
        base = pl.multiple_of(wid * BPW, BPW)
        nbase = pl.multiple_of(wid * (BPW * NEG), BPW * NEG)
        pltpu.sync_copy(ctr_hbm.at[pl.ds(base, BPW)], ctr_idx)
        pltpu.sync_copy(ctx_hbm.at[pl.ds(base, BPW)], ctx_idx)
        pltpu.sync_copy(neg_hbm.at[pl.ds(nbase, BPW * NEG)], neg_idx)

        lanes = lax.iota(jnp.int32, L)
        lane0 = lanes == 0
        perms = [lanes ^ sh for sh in (8, 4, 2, 1)]

        def transfers(t, bset, phase):
            ctr_b, ctx_b, neg_b = bset
            u_tab, v_tab = (ug_hbm, vg_hbm) if phase == 0 else (ur_hbm, vr_hbm)
            toff = pl.multiple_of(t * BB, BB)
            pairs = [
                (u_tab.at[ctr_idx.at[pl.ds(toff, BB)]], ctr_b),
                (v_tab.at[ctx_idx.at[pl.ds(toff, BB)]], ctx_b),
            ]
            noff = pl.multiple_of(t * NROWS, CHUNK)
            for c in range(NCH):
                src = neg_idx.at[pl.ds(noff + c * CHUNK, CHUNK)]
                pairs.append((v_tab.at[src], neg_b.at[pl.ds(c * CHUNK, CHUNK)]))
            return pairs

        def issue(t, s, phase):
            for src, dst in transfers(t, bufs[s], phase):
                pltpu.async_copy(src, dst, sems[s], add=(phase == 1))

        def drain(t, s, phase):
            for src, dst in transfers(t, bufs[s], phase):
                pltpu.make_async_copy(src, dst, sems[s]).wait()

        shuffle_dn = lax.GatherDimensionNumbers(
            offset_dims=(), collapsed_slice_dims=(0,), start_index_map=(0,))

        def reduce_full(acc):
            for p in perms:
                acc = acc + lax.gather(
                    acc, p[:, None], shuffle_dn, slice_sizes=(1,),
                    mode=lax.GatherScatterMode.PROMISE_IN_BOUNDS)
            return acc

        def compute(t, s):
            ctr_b, ctx_b, neg_b = bufs[s]

            def bbody(b, carry):
                c = [ctr_b[b, pl.ds(16 * j, 16)] for j in range(NV)]
                col = jnp.full((L,), t * BB, jnp.int32) + b

                def emit(k, acc):
                    plsc.store_scatter(
                        scores, [jnp.full((L,), k, jnp.int32), col],
                        reduce_full(acc), mask=lane0)

                acc = ctx_b[b, pl.ds(0, 16)] * c[0]
                for j in range(1, NV):
                    acc += ctx_b[b, pl.ds(16 * j, 16)] * c[j]
                emit(0, acc)
                for k in range(NEG):
                    row = b * NEG + k
                    acc = neg_b[row, pl.ds(0, 16)] * c[0]
                    for j in range(1, NV):
                        acc += neg_b[row, pl.ds(16 * j, 16)] * c[j]
                    emit(k + 1, acc)
                return carry

            lax.fori_loop(0, BB, bbody, 0)

        issue(0, 0, 0)
        issue(1, 1, 0)

        def tbody(tt, carry):
            t = tt * 2

            def stage(t, s):
                drain(t, s, 0)
                issue(t, s, 1)
                drain(t, s, 1)
                compute(t, s)

                @pl.when(t + 2 < NT)
                def _():
                    issue(t + 2, s, 0)

            stage(t, 0)
            stage(t + 1, 1)
            return carry

        lax.fori_loop(0, NT // 2, tbody, 0)

        pltpu.sync_copy(scores, out_hbm.at[:, pl.ds(base, BPW)])

    return body(ctr_ids, ctx_ids, neg_flat, u_global, v_global, u_reg, v_reg)


def _tc_loss(scores):
    def body(s_ref, o_ref):
        s = s_ref[...]

        def logsig(x):
            return jnp.minimum(x, 0.0) - jnp.log1p(jnp.exp(-jnp.abs(x)))

        p = logsig(jnp.clip(s[0:1, :], -10.0, 10.0))
        n = logsig(-jnp.clip(s[1:1 + NEG, :], -10.0, 10.0))
        o_ref[0, 0] = -(jnp.sum(p) + jnp.sum(n)) / BATCH

    return pl.pallas_call(
        body,
        out_shape=jax.ShapeDtypeStruct((1, 1), jnp.float32),
        out_specs=pl.BlockSpec(memory_space=pltpu.SMEM),
    )(scores)


def kernel(center_ids, context_ids, neg_ids, u_global, v_global, u_reg, v_reg):
    ctr = center_ids.astype(jnp.int32)
    ctx = context_ids.astype(jnp.int32)
    neg = neg_ids.astype(jnp.int32).reshape(-1)
    scores = _sc_scores(ctr, ctx, neg, u_global, v_global, u_reg, v_reg)
    return _tc_loss(scores)[0, 0]


# speedup vs baseline: 1.3509x; 1.3509x over previous
import functools

import jax
import jax.numpy as jnp
from jax import lax
from jax.experimental import pallas as pl
from jax.experimental.pallas import tpu as pltpu
from jax.experimental.pallas import tpu_sc as plsc

EMB = 128
BATCH = 4096
NEG = 20
L = 16
NV = EMB // L
NC, NS = 2, 16
NW = NC * NS
BPW = BATCH // NW
BB = 8
NT = BPW // BB
NROWS = BB * NEG
CHUNK = 80
NCH = NROWS // CHUNK


def _sc_scores(ctr_ids, ctx_ids, neg_flat, u_global, v_global, u_reg, v_reg):
    mesh = plsc.VectorSubcoreMesh(
        core_axis_name="c", subcore_axis_name="s", num_cores=NC, num_subcores=NS
    )

    row_buf = lambda n: pltpu.VMEM((n, EMB), jnp.float32)
    set_bufs = [row_buf(BB), row_buf(BB), row_buf(NROWS)]

    @functools.partial(
        pl.kernel,
        out_type=jax.ShapeDtypeStruct((1 + NEG, BATCH), jnp.float32),
        mesh=mesh,
        compiler_params=pltpu.CompilerParams(needs_layout_passes=False),
        scratch_types=[
            pltpu.VMEM((BPW,), jnp.int32),
            pltpu.VMEM((BPW,), jnp.int32),
            pltpu.VMEM((BPW * NEG,), jnp.int32),
            *set_bufs,
            *set_bufs,
            *set_bufs,
            pltpu.VMEM((1 + NEG, BPW), jnp.float32),
            pltpu.SemaphoreType.DMA,
            pltpu.SemaphoreType.DMA,
            pltpu.SemaphoreType.DMA,
        ],
    )
    def body(ctr_hbm, ctx_hbm, neg_hbm, ug_hbm, vg_hbm, ur_hbm, vr_hbm, out_hbm,
             ctr_idx, ctx_idx, neg_idx, *rest):
        bufs = (rest[0:3], rest[3:6], rest[6:9])
        scores = rest[9]
        sems = (rest[10], rest[11], rest[12])
        wid = lax.axis_index("s") * NC + lax.axis_index("c")
        base = pl.multiple_of(wid * BPW, BPW)
        nbase = pl.multiple_of(wid * (BPW * NEG), BPW * NEG)
        pltpu.sync_copy(ctr_hbm.at[pl.ds(base, BPW)], ctr_idx)
        pltpu.sync_copy(ctx_hbm.at[pl.ds(base, BPW)], ctx_idx)
        pltpu.sync_copy(neg_hbm.at[pl.ds(nbase, BPW * NEG)], neg_idx)

        lanes = lax.iota(jnp.int32, L)
        lane0 = lanes == 0
        perms = [lanes ^ sh for sh in (8, 4, 2, 1)]

        def transfers(t, bset, phase):
            ctr_b, ctx_b, neg_b = bset
            u_tab, v_tab = (ug_hbm, vg_hbm) if phase == 0 else (ur_hbm, vr_hbm)
            toff = pl.multiple_of(t * BB, BB)
            pairs = [
                (u_tab.at[ctr_idx.at[pl.ds(toff, BB)]], ctr_b),
                (v_tab.at[ctx_idx.at[pl.ds(toff, BB)]], ctx_b),
            ]
            noff = pl.multiple_of(t * NROWS, CHUNK)
            for c in range(NCH):
                src = neg_idx.at[pl.ds(noff + c * CHUNK, CHUNK)]
                pairs.append((v_tab.at[src], neg_b.at[pl.ds(c * CHUNK, CHUNK)]))
            return pairs

        def issue(t, s, phase):
            for src, dst in transfers(t, bufs[s], phase):
                pltpu.async_copy(src, dst, sems[s], add=(phase == 1))

        def drain(t, s, phase):
            for src, dst in transfers(t, bufs[s], phase):
                pltpu.make_async_copy(src, dst, sems[s]).wait()

        shuffle_dn = lax.GatherDimensionNumbers(
            offset_dims=(), collapsed_slice_dims=(0,), start_index_map=(0,))

        def reduce_full(acc):
            for p in perms:
                acc = acc + lax.gather(
                    acc, p[:, None], shuffle_dn, slice_sizes=(1,),
                    mode=lax.GatherScatterMode.PROMISE_IN_BOUNDS)
            return acc

        def compute(t, s):
            ctr_b, ctx_b, neg_b = bufs[s]

            def bbody(b, carry):
                c = [ctr_b[b, pl.ds(16 * j, 16)] for j in range(NV)]
                col = jnp.full((L,), t * BB, jnp.int32) + b

                def emit(k, acc):
                    plsc.store_scatter(
                        scores, [jnp.full((L,), k, jnp.int32), col],
                        reduce_full(acc), mask=lane0)

                acc = ctx_b[b, pl.ds(0, 16)] * c[0]
                for j in range(1, NV):
                    acc += ctx_b[b, pl.ds(16 * j, 16)] * c[j]
                emit(0, acc)
                for k in range(NEG):
                    row = b * NEG + k
                    acc = neg_b[row, pl.ds(0, 16)] * c[0]
                    for j in range(1, NV):
                        acc += neg_b[row, pl.ds(16 * j, 16)] * c[j]
                    emit(k + 1, acc)
                return carry

            lax.fori_loop(0, BB, bbody, 0)

        issue(0, 0, 0)
        issue(1, 1, 0)
        issue(2, 2, 0)
        drain(0, 0, 0)
        issue(0, 0, 1)

        def tbody(tt, carry):
            for i in range(3):
                t = tt * 3 + i
                s = i

                @pl.when(t < NT)
                def _(t=t, s=s):
                    @pl.when(t + 1 < NT)
                    def _():
                        drain(t + 1, (s + 1) % 3, 0)
                        issue(t + 1, (s + 1) % 3, 1)

                    drain(t, s, 1)
                    compute(t, s)

                    @pl.when(t + 3 < NT)
                    def _():
                        issue(t + 3, s, 0)

            return carry

        lax.fori_loop(0, (NT + 2) // 3, tbody, 0)

        pltpu.sync_copy(scores, out_hbm.at[:, pl.ds(base, BPW)])

    return body(ctr_ids, ctx_ids, neg_flat, u_global, v_global, u_reg, v_reg)


def _tc_loss(scores):
    def body(s_ref, o_ref):
        s = s_ref[...]

        def logsig(x):
            return jnp.minimum(x, 0.0) - jnp.log1p(jnp.exp(-jnp.abs(x)))

        p = logsig(jnp.clip(s[0:1, :], -10.0, 10.0))
        n = logsig(-jnp.clip(s[1:1 + NEG, :], -10.0, 10.0))
        o_ref[0, 0] = -(jnp.sum(p) + jnp.sum(n)) / BATCH

    return pl.pallas_call(
        body,
        out_shape=jax.ShapeDtypeStruct((1, 1), jnp.float32),
        out_specs=pl.BlockSpec(memory_space=pltpu.SMEM),
    )(scores)


def kernel(center_ids, context_ids, neg_ids, u_global, v_global, u_reg, v_reg):
    ctr = center_ids.astype(jnp.int32)
    ctx = context_ids.astype(jnp.int32)
    neg = neg_ids.astype(jnp.int32).reshape(-1)
    scores = _sc_scores(ctr, ctx, neg, u_global, v_global, u_reg, v_reg)
    return _tc_loss(scores)[0, 0]


# speedup vs baseline: 1.9044x; 1.4098x over previous
import functools

import jax
import jax.numpy as jnp
from jax import lax
from jax.experimental import pallas as pl
from jax.experimental.pallas import tpu as pltpu
from jax.experimental.pallas import tpu_sc as plsc

EMB = 128
BATCH = 4096
NEG = 20
L = 16
NV = EMB // L
NC, NS = 2, 16
NW = NC * NS
BPW = BATCH // NW
BB = 8
NT = BPW // BB
NROWS = BB * NEG
CHUNK = 80
NCH = NROWS // CHUNK


def _sc_scores(ctr_ids, ctx_ids, neg_flat, u_global, v_global, u_reg, v_reg):
    mesh = plsc.VectorSubcoreMesh(
        core_axis_name="c", subcore_axis_name="s", num_cores=NC, num_subcores=NS
    )

    row_buf = lambda n: pltpu.VMEM((n, EMB), jnp.float32)
    set_bufs = [row_buf(BB), row_buf(BB), row_buf(NROWS)]

    @functools.partial(
        pl.kernel,
        out_type=jax.ShapeDtypeStruct((NW, L), jnp.float32),
        mesh=mesh,
        compiler_params=pltpu.CompilerParams(needs_layout_passes=False),
        scratch_types=[
            pltpu.VMEM((BPW,), jnp.int32),
            pltpu.VMEM((BPW,), jnp.int32),
            pltpu.VMEM((BPW * NEG,), jnp.int32),
            *set_bufs,
            *set_bufs,
            *set_bufs,
            pltpu.VMEM((L,), jnp.float32),
            pltpu.SemaphoreType.DMA,
            pltpu.SemaphoreType.DMA,
            pltpu.SemaphoreType.DMA,
        ],
    )
    def body(ctr_hbm, ctx_hbm, neg_hbm, ug_hbm, vg_hbm, ur_hbm, vr_hbm, out_hbm,
             ctr_idx, ctx_idx, neg_idx, *rest):
        bufs = (rest[0:3], rest[3:6], rest[6:9])
        lacc = rest[9]
        sems = (rest[10], rest[11], rest[12])
        wid = lax.axis_index("s") * NC + lax.axis_index("c")
        base = pl.multiple_of(wid * BPW, BPW)
        nbase = pl.multiple_of(wid * (BPW * NEG), BPW * NEG)
        pltpu.sync_copy(ctr_hbm.at[pl.ds(base, BPW)], ctr_idx)
        pltpu.sync_copy(ctx_hbm.at[pl.ds(base, BPW)], ctx_idx)
        pltpu.sync_copy(neg_hbm.at[pl.ds(nbase, BPW * NEG)], neg_idx)

        lanes = lax.iota(jnp.int32, L)
        perms = [lanes ^ sh for sh in (8, 4, 2, 1)]
        lacc[...] = jnp.zeros((L,), jnp.float32)

        def logsig(x):
            t = jnp.exp(-jnp.abs(x))
            z = t / (2.0 + t)
            z2 = z * z
            l1p = 2.0 * z * (1.0 + z2 * (1.0 / 3.0 + z2 * (0.2 + z2 / 7.0)))
            return jnp.minimum(x, 0.0) - l1p

        def transfers(t, bset, phase):
            ctr_b, ctx_b, neg_b = bset
            u_tab, v_tab = (ug_hbm, vg_hbm) if phase == 0 else (ur_hbm, vr_hbm)
            toff = pl.multiple_of(t * BB, BB)
            pairs = [
                (u_tab.at[ctr_idx.at[pl.ds(toff, BB)]], ctr_b),
                (v_tab.at[ctx_idx.at[pl.ds(toff, BB)]], ctx_b),
            ]
            noff = pl.multiple_of(t * NROWS, CHUNK)
            for c in range(NCH):
                src = neg_idx.at[pl.ds(noff + c * CHUNK, CHUNK)]
                pairs.append((v_tab.at[src], neg_b.at[pl.ds(c * CHUNK, CHUNK)]))
            return pairs

        def issue(t, s, phase):
            for src, dst in transfers(t, bufs[s], phase):
                pltpu.async_copy(src, dst, sems[s], add=(phase == 1))

        def drain(t, s, phase):
            for src, dst in transfers(t, bufs[s], phase):
                pltpu.make_async_copy(src, dst, sems[s]).wait()

        shuffle_dn = lax.GatherDimensionNumbers(
            offset_dims=(), collapsed_slice_dims=(0,), start_index_map=(0,))

        def reduce_full(acc):
            for p in perms:
                acc = acc + lax.gather(
                    acc, p[:, None], shuffle_dn, slice_sizes=(1,),
                    mode=lax.GatherScatterMode.PROMISE_IN_BOUNDS)
            return acc

        def compute(t, s):
            ctr_b, ctx_b, neg_b = bufs[s]

            def bbody(b, loss):
                c = [ctr_b[b, pl.ds(16 * j, 16)] for j in range(NV)]

                acc = ctx_b[b, pl.ds(0, 16)] * c[0]
                for j in range(1, NV):
                    acc += ctx_b[b, pl.ds(16 * j, 16)] * c[j]
                pos = jnp.clip(reduce_full(acc), -10.0, 10.0)
                loss = loss + logsig(pos)
                for k in range(NEG):
                    row = b * NEG + k
                    acc = neg_b[row, pl.ds(0, 16)] * c[0]
                    for j in range(1, NV):
                        acc += neg_b[row, pl.ds(16 * j, 16)] * c[j]
                    ns = jnp.clip(reduce_full(acc), -10.0, 10.0)
                    loss = loss + logsig(-ns)
                return loss

            tile_loss = lax.fori_loop(
                0, BB, bbody, jnp.zeros((L,), jnp.float32))
            lacc[...] = lacc[...] + tile_loss

        issue(0, 0, 0)
        issue(1, 1, 0)
        issue(2, 2, 0)
        drain(0, 0, 0)
        issue(0, 0, 1)

        def tbody(tt, carry):
            for i in range(3):
                t = tt * 3 + i
                s = i

                @pl.when(t < NT)
                def _(t=t, s=s):
                    @pl.when(t + 1 < NT)
                    def _():
                        drain(t + 1, (s + 1) % 3, 0)
                        issue(t + 1, (s + 1) % 3, 1)

                    drain(t, s, 1)
                    compute(t, s)

                    @pl.when(t + 3 < NT)
                    def _():
                        issue(t + 3, s, 0)

            return carry

        lax.fori_loop(0, (NT + 2) // 3, tbody, 0)

        pltpu.sync_copy(lacc, out_hbm.at[wid])

    return body(ctr_ids, ctx_ids, neg_flat, u_global, v_global, u_reg, v_reg)


def _tc_loss(partials):
    def body(s_ref, o_ref):
        o_ref[0, 0] = -jnp.sum(s_ref[...][:, 0]) / BATCH

    return pl.pallas_call(
        body,
        out_shape=jax.ShapeDtypeStruct((1, 1), jnp.float32),
        out_specs=pl.BlockSpec(memory_space=pltpu.SMEM),
    )(partials)


def kernel(center_ids, context_ids, neg_ids, u_global, v_global, u_reg, v_reg):
    ctr = center_ids.astype(jnp.int32)
    ctx = context_ids.astype(jnp.int32)
    neg = neg_ids.astype(jnp.int32).reshape(-1)
    scores = _sc_scores(ctr, ctx, neg, u_global, v_global, u_reg, v_reg)
    return _tc_loss(scores)[0, 0]


# speedup vs baseline: 1.9951x; 1.0476x over previous
import functools

import jax
import jax.numpy as jnp
from jax import lax
from jax.experimental import pallas as pl
from jax.experimental.pallas import tpu as pltpu
from jax.experimental.pallas import tpu_sc as plsc

EMB = 128
BATCH = 4096
NEG = 20
L = 16
NV = EMB // L
NC, NS = 2, 16
NW = NC * NS
BPW = BATCH // NW
BB = 8
NT = BPW // BB
NROWS = BB * NEG
CHUNK = 80
NCH = NROWS // CHUNK


def _sc_scores(ctr_ids, ctx_ids, neg_flat, u_global, v_global, u_reg, v_reg):
    mesh = plsc.VectorSubcoreMesh(
        core_axis_name="c", subcore_axis_name="s", num_cores=NC, num_subcores=NS
    )

    row_buf = lambda n: pltpu.VMEM((n, EMB), jnp.float32)
    set_bufs = [row_buf(BB), row_buf(BB), row_buf(NROWS)]

    @functools.partial(
        pl.kernel,
        out_type=jax.ShapeDtypeStruct((NW, L), jnp.float32),
        mesh=mesh,
        compiler_params=pltpu.CompilerParams(needs_layout_passes=False),
        scratch_types=[
            pltpu.VMEM((BPW,), jnp.int32),
            pltpu.VMEM((BPW,), jnp.int32),
            pltpu.VMEM((BPW * NEG,), jnp.int32),
            *set_bufs,
            *set_bufs,
            *set_bufs,
            pltpu.VMEM((L,), jnp.float32),
            pltpu.SemaphoreType.DMA,
            pltpu.SemaphoreType.DMA,
            pltpu.SemaphoreType.DMA,
        ],
    )
    def body(ctr_hbm, ctx_hbm, neg_hbm, ug_hbm, vg_hbm, ur_hbm, vr_hbm, out_hbm,
             ctr_idx, ctx_idx, neg_idx, *rest):
        bufs = (rest[0:3], rest[3:6], rest[6:9])
        lacc = rest[9]
        sems = (rest[10], rest[11], rest[12])
        wid = lax.axis_index("s") * NC + lax.axis_index("c")
        base = pl.multiple_of(wid * BPW, BPW)
        nbase = pl.multiple_of(wid * (BPW * NEG), BPW * NEG)
        pltpu.sync_copy(ctr_hbm.at[pl.ds(base, BPW)], ctr_idx)
        pltpu.sync_copy(ctx_hbm.at[pl.ds(base, BPW)], ctx_idx)
        pltpu.sync_copy(neg_hbm.at[pl.ds(nbase, BPW * NEG)], neg_idx)

        lanes = lax.iota(jnp.int32, L)
        perms = [lanes ^ sh for sh in (8, 4, 2, 1)]
        sign0 = jnp.where(lanes == 0, 1.0, -1.0).astype(jnp.float32)
        low5 = lanes < (1 + NEG - L)
        zeros = jnp.zeros((L,), jnp.float32)
        lacc[...] = zeros

        def logsig(x):
            t = jnp.exp(-jnp.abs(x))
            z = t / (2.0 + t)
            z2 = z * z
            l1p = 2.0 * z * (1.0 + z2 * (1.0 / 3.0 + z2 * (0.2 + z2 / 7.0)))
            return jnp.minimum(x, 0.0) - l1p

        def transfers(t, bset, phase):
            ctr_b, ctx_b, neg_b = bset
            u_tab, v_tab = (ug_hbm, vg_hbm) if phase == 0 else (ur_hbm, vr_hbm)
            toff = pl.multiple_of(t * BB, BB)
            pairs = [
                (u_tab.at[ctr_idx.at[pl.ds(toff, BB)]], ctr_b),
                (v_tab.at[ctx_idx.at[pl.ds(toff, BB)]], ctx_b),
            ]
            noff = pl.multiple_of(t * NROWS, CHUNK)
            for c in range(NCH):
                src = neg_idx.at[pl.ds(noff + c * CHUNK, CHUNK)]
                pairs.append((v_tab.at[src], neg_b.at[pl.ds(c * CHUNK, CHUNK)]))
            return pairs

        def issue(t, s, phase):
            for src, dst in transfers(t, bufs[s], phase):
                pltpu.async_copy(src, dst, sems[s], add=(phase == 1))

        def drain(t, s, phase):
            for src, dst in transfers(t, bufs[s], phase):
                pltpu.make_async_copy(src, dst, sems[s]).wait()

        shuffle_dn = lax.GatherDimensionNumbers(
            offset_dims=(), collapsed_slice_dims=(0,), start_index_map=(0,))

        def reduce_full(acc):
            for p in perms:
                acc = acc + lax.gather(
                    acc, p[:, None], shuffle_dn, slice_sizes=(1,),
                    mode=lax.GatherScatterMode.PROMISE_IN_BOUNDS)
            return acc

        def compute(t, s):
            ctr_b, ctx_b, neg_b = bufs[s]

            def one_b(b, loss):
                c = [ctr_b[b, pl.ds(16 * j, 16)] for j in range(NV)]

                def dot(buf, row):
                    even = buf[row, pl.ds(0, 16)] * c[0]
                    odd = buf[row, pl.ds(16, 16)] * c[1]
                    for j in range(2, NV, 2):
                        even += buf[row, pl.ds(16 * j, 16)] * c[j]
                    for j in range(3, NV, 2):
                        odd += buf[row, pl.ds(16 * j, 16)] * c[j]
                    return reduce_full(even + odd)

                p0 = dot(ctx_b, b)
                row0 = b * NEG
                for k in range(L - 1):
                    p0 = jnp.where(lanes == (k + 1), dot(neg_b, row0 + k), p0)
                p1 = zeros
                for k in range(L - 1, NEG):
                    p1 = jnp.where(lanes == (k - (L - 1)),
                                   dot(neg_b, row0 + k), p1)
                x0 = jnp.clip(p0 * sign0, -10.0, 10.0)
                x1 = jnp.clip(-p1, -10.0, 10.0)
                return loss + logsig(x0) + jnp.where(low5, logsig(x1), 0.0)

            def bbody(i, carry):
                l0, l1 = carry
                return (one_b(i * 2, l0), one_b(i * 2 + 1, l1))

            l0, l1 = lax.fori_loop(0, BB // 2, bbody, (zeros, zeros))
            lacc[...] = lacc[...] + l0 + l1

        issue(0, 0, 0)
        issue(1, 1, 0)
        issue(2, 2, 0)
        drain(0, 0, 0)
        issue(0, 0, 1)

        def tbody(tt, carry):
            for i in range(3):
                t = tt * 3 + i
                s = i

                @pl.when(t < NT)
                def _(t=t, s=s):
                    @pl.when(t + 1 < NT)
                    def _():
                        drain(t + 1, (s + 1) % 3, 0)
                        issue(t + 1, (s + 1) % 3, 1)

                    drain(t, s, 1)
                    compute(t, s)

                    @pl.when(t + 3 < NT)
                    def _():
                        issue(t + 3, s, 0)

            return carry

        lax.fori_loop(0, (NT + 2) // 3, tbody, 0)

        lacc[...] = reduce_full(lacc[...])
        pltpu.sync_copy(lacc, out_hbm.at[wid])

    return body(ctr_ids, ctx_ids, neg_flat, u_global, v_global, u_reg, v_reg)


def _tc_loss(partials):
    def body(s_ref, o_ref):
        o_ref[0, 0] = -jnp.sum(s_ref[...][:, 0]) / BATCH

    return pl.pallas_call(
        body,
        out_shape=jax.ShapeDtypeStruct((1, 1), jnp.float32),
        out_specs=pl.BlockSpec(memory_space=pltpu.SMEM),
    )(partials)


def kernel(center_ids, context_ids, neg_ids, u_global, v_global, u_reg, v_reg):
    ctr = center_ids.astype(jnp.int32)
    ctx = context_ids.astype(jnp.int32)
    neg = neg_ids.astype(jnp.int32).reshape(-1)
    scores = _sc_scores(ctr, ctx, neg, u_global, v_global, u_reg, v_reg)
    return _tc_loss(scores)[0, 0]
